# Initial kernel scaffold; baseline (speedup 1.0000x reference)
import functools

import jax
import jax.numpy as jnp
from jax.experimental import pallas as pl
from jax.experimental.pallas import tpu as pltpu

INPUT_DIM = 1024
LATENT_DIM = 4096
K = 64
TM = 256


def _body(x_ref, we_ref, be_ref, wd_ref, bd_ref, sp_ref, rec_ref):
    lat = jax.lax.dot_general(
        x_ref[...], we_ref[...], (((1,), (1,)), ((), ())),
        preferred_element_type=jnp.float32,
    ) + be_ref[...]

    ikey = jax.lax.bitcast_convert_type(lat, jnp.int32)
    key = jnp.where(ikey < 0, ikey ^ jnp.int32(0x7FFFFFFF), ikey)

    t0 = jnp.full((TM, 1), jnp.iinfo(jnp.int32).min, dtype=jnp.int32)

    def step(j, t):
        bit = 30 - j
        cand = t + jax.lax.shift_left(jnp.int32(1), bit)
        cnt = jnp.sum((key >= cand).astype(jnp.int32), axis=1, keepdims=True)
        return jnp.where(cnt >= K, cand, t)

    t = jax.lax.fori_loop(0, 31, step, t0)

    sparse = jnp.where(key >= t, lat, 0.0)
    sp_ref[...] = sparse

    rec = jax.lax.dot_general(
        sparse, wd_ref[...], (((1,), (1,)), ((), ())),
        preferred_element_type=jnp.float32,
    ) + bd_ref[...]
    rec_ref[...] = rec


@jax.jit
def kernel(x, W_enc, b_enc, W_dec, b_dec):
    B = x.shape[0]
    grid = (B // TM,)
    out = pl.pallas_call(
        _body,
        grid=grid,
        in_specs=[
            pl.BlockSpec((TM, INPUT_DIM), lambda i: (i, 0)),
            pl.BlockSpec((LATENT_DIM, INPUT_DIM), lambda i: (0, 0)),
            pl.BlockSpec((1, LATENT_DIM), lambda i: (0, 0)),
            pl.BlockSpec((INPUT_DIM, LATENT_DIM), lambda i: (0, 0)),
            pl.BlockSpec((1, INPUT_DIM), lambda i: (0, 0)),
        ],
        out_specs=[
            pl.BlockSpec((TM, LATENT_DIM), lambda i: (i, 0)),
            pl.BlockSpec((TM, INPUT_DIM), lambda i: (i, 0)),
        ],
        out_shape=[
            jax.ShapeDtypeStruct((B, LATENT_DIM), jnp.float32),
            jax.ShapeDtypeStruct((B, INPUT_DIM), jnp.float32),
        ],
    )(x, W_enc, b_enc.reshape(1, LATENT_DIM), W_dec, b_dec.reshape(1, INPUT_DIM))
    sparse, recon = out
    return (recon, sparse)


# speedup vs baseline: 13.6457x; 13.6457x over previous
import functools

import jax
import jax.numpy as jnp
from jax.experimental import pallas as pl
from jax.experimental.pallas import tpu as pltpu

INPUT_DIM = 1024
LATENT_DIM = 4096
K = 64
TM = 256


def _body(x_ref, we_ref, be_ref, wd_ref, bd_ref, sp_ref, rec_ref):
    lat = jax.lax.dot_general(
        x_ref[...], we_ref[...], (((1,), (1,)), ((), ())),
        preferred_element_type=jnp.float32,
    ) + be_ref[...]

    ikey = jax.lax.bitcast_convert_type(lat, jnp.int32)
    key = jnp.where(ikey < 0, ikey ^ jnp.int32(0x7FFFFFFF), ikey)

    cnt0 = jnp.sum((key >= 0).astype(jnp.int32), axis=1, keepdims=True)
    t0 = jnp.where(cnt0 >= K, jnp.int32(0), jnp.iinfo(jnp.int32).min)

    def step(j, t):
        bit = 30 - j
        cand = t + jax.lax.shift_left(jnp.int32(1), bit)
        cnt = jnp.sum((key >= cand).astype(jnp.int32), axis=1, keepdims=True)
        return jnp.where(cnt >= K, cand, t)

    t = jax.lax.fori_loop(0, 31, step, t0)

    sparse = jnp.where(key >= t, lat, 0.0)
    sp_ref[...] = sparse

    rec = jax.lax.dot_general(
        sparse, wd_ref[...], (((1,), (1,)), ((), ())),
        preferred_element_type=jnp.float32,
    ) + bd_ref[...]
    rec_ref[...] = rec


@jax.jit
def kernel(x, W_enc, b_enc, W_dec, b_dec):
    B = x.shape[0]
    grid = (B // TM,)
    out = pl.pallas_call(
        _body,
        grid=grid,
        in_specs=[
            pl.BlockSpec((TM, INPUT_DIM), lambda i: (i, 0)),
            pl.BlockSpec((LATENT_DIM, INPUT_DIM), lambda i: (0, 0)),
            pl.BlockSpec((1, LATENT_DIM), lambda i: (0, 0)),
            pl.BlockSpec((INPUT_DIM, LATENT_DIM), lambda i: (0, 0)),
            pl.BlockSpec((1, INPUT_DIM), lambda i: (0, 0)),
        ],
        out_specs=[
            pl.BlockSpec((TM, LATENT_DIM), lambda i: (i, 0)),
            pl.BlockSpec((TM, INPUT_DIM), lambda i: (i, 0)),
        ],
        out_shape=[
            jax.ShapeDtypeStruct((B, LATENT_DIM), jnp.float32),
            jax.ShapeDtypeStruct((B, INPUT_DIM), jnp.float32),
        ],
    )(x, W_enc, b_enc.reshape(1, LATENT_DIM), W_dec, b_dec.reshape(1, INPUT_DIM))
    sparse, recon = out
    return (recon, sparse)
